# baseline (speedup 1.0000x reference)
import jax
import jax.numpy as jnp
from jax.experimental import pallas as pl
from jax.experimental.pallas import tpu as pltpu

_N = 10000
_D = 128
_K = 20
_RB = 400
_NBLK = _N // _RB
_LANES = 128
_NSLAB = 79
_NPAD = _NSLAB * _LANES
_NEG = -3.0


def _prep_body(x_ref, o_ref):
    x = x_ref[...]
    s = jnp.sum(x * x, axis=1, keepdims=True)
    y = x / jnp.sqrt(jnp.maximum(s, 1e-12))
    o_ref[...] = y.astype(jnp.bfloat16)


def _topk_body(rows_ref, all_ref, vals_ref, idxs_ref, s_ref):
    i = pl.program_id(0)
    a = rows_ref[...]
    b = all_ref[...]
    sim = jax.lax.dot_general(
        a, b, (((1,), (1,)), ((), ())), preferred_element_type=jnp.float32
    )
    col = jax.lax.broadcasted_iota(jnp.int32, (_RB, _N), 1)
    row = jax.lax.broadcasted_iota(jnp.int32, (_RB, _N), 0) + i * _RB
    sim = jnp.where(col == row, -2.0, sim)
    s_ref[...] = jnp.concatenate(
        [sim, jnp.full((_RB, _NPAD - _N), _NEG, jnp.float32)], axis=1
    )

    lane = jax.lax.broadcasted_iota(jnp.int32, (_RB, _LANES), 1)
    w = jnp.full((_RB, 1), 2.0, jnp.float32)
    cthr = jnp.full((_RB, 1), -1, jnp.int32)
    vals = []
    idxs = []
    for _ in range(_K):

        def sweep(c, carry):
            rmax, rarg = carry
            x = s_ref[:, pl.ds(pl.multiple_of(c * _LANES, _LANES), _LANES)]
            rem = (x < w) | ((x == w) & (lane > cthr - c * _LANES))
            xm = jnp.where(rem, x, _NEG)
            gt = xm > rmax
            rarg = jnp.where(gt, c, rarg)
            rmax = jnp.maximum(xm, rmax)
            return rmax, rarg

        rmax, rarg = jax.lax.fori_loop(
            0,
            _NSLAB,
            sweep,
            (
                jnp.full((_RB, _LANES), _NEG - 1.0, jnp.float32),
                jnp.zeros((_RB, _LANES), jnp.int32),
            ),
        )
        m = jnp.max(rmax, axis=1, keepdims=True)
        cand = jnp.where(rmax == m, rarg * _LANES + lane, _NPAD)
        idx = jnp.min(cand, axis=1, keepdims=True)
        vals.append(m)
        idxs.append(idx)
        w = m
        cthr = idx
    vals_ref[...] = jnp.concatenate(vals, axis=1)
    idxs_ref[...] = jnp.concatenate(idxs, axis=1)


def kernel(embeddings):
    norm_bf16 = pl.pallas_call(
        _prep_body,
        out_shape=jax.ShapeDtypeStruct((_N, _D), jnp.bfloat16),
    )(embeddings)

    vals, idxs = pl.pallas_call(
        _topk_body,
        grid=(_NBLK,),
        in_specs=[
            pl.BlockSpec((_RB, _D), lambda i: (i, 0)),
            pl.BlockSpec((_N, _D), lambda i: (0, 0)),
        ],
        out_specs=[
            pl.BlockSpec((_RB, _K), lambda i: (i, 0)),
            pl.BlockSpec((_RB, _K), lambda i: (i, 0)),
        ],
        out_shape=[
            jax.ShapeDtypeStruct((_N, _K), jnp.float32),
            jax.ShapeDtypeStruct((_N, _K), jnp.int32),
        ],
        scratch_shapes=[pltpu.VMEM((_RB, _NPAD), jnp.float32)],
        compiler_params=pltpu.CompilerParams(
            dimension_semantics=("arbitrary",),
        ),
    )(norm_bf16, norm_bf16)
    return vals, idxs


# speedup vs baseline: 1.6643x; 1.6643x over previous
import jax
import jax.numpy as jnp
from jax.experimental import pallas as pl
from jax.experimental.pallas import tpu as pltpu

_N = 10000
_D = 128
_K = 20
_RB = 400
_NBLK = _N // _RB
_NEG = -3.0


def _prep_body(x_ref, o_ref):
    x = x_ref[...]
    s = jnp.sum(x * x, axis=1, keepdims=True)
    y = x / jnp.sqrt(jnp.maximum(s, 1e-12))
    o_ref[...] = y.astype(jnp.bfloat16)


def _topk_body(rows_ref, all_ref, vals_ref, idxs_ref):
    i = pl.program_id(0)
    a = rows_ref[...]
    b = all_ref[...]
    sim = jax.lax.dot_general(
        a, b, (((1,), (1,)), ((), ())), preferred_element_type=jnp.float32
    )
    col = jax.lax.broadcasted_iota(jnp.int32, (_RB, _N), 1)
    row = jax.lax.broadcasted_iota(jnp.int32, (_RB, _N), 0) + i * _RB
    s = jnp.where(col == row, -2.0, sim)
    w = jnp.full((_RB, 1), 2.0, jnp.float32)
    ci = jnp.full((_RB, 1), -1, jnp.int32)
    vals = []
    idxs = []
    for _ in range(_K):
        sm = jnp.where((s < w) | ((s == w) & (col > ci)), s, _NEG)
        m = jnp.max(sm, axis=1, keepdims=True)
        idx = jnp.min(jnp.where(sm == m, col, _N), axis=1, keepdims=True)
        vals.append(m)
        idxs.append(idx)
        w = m
        ci = idx
    vals_ref[...] = jnp.concatenate(vals, axis=1)
    idxs_ref[...] = jnp.concatenate(idxs, axis=1)


def kernel(embeddings):
    norm_bf16 = pl.pallas_call(
        _prep_body,
        out_shape=jax.ShapeDtypeStruct((_N, _D), jnp.bfloat16),
    )(embeddings)

    vals, idxs = pl.pallas_call(
        _topk_body,
        grid=(_NBLK,),
        in_specs=[
            pl.BlockSpec((_RB, _D), lambda i: (i, 0)),
            pl.BlockSpec((_N, _D), lambda i: (0, 0)),
        ],
        out_specs=[
            pl.BlockSpec((_RB, _K), lambda i: (i, 0)),
            pl.BlockSpec((_RB, _K), lambda i: (i, 0)),
        ],
        out_shape=[
            jax.ShapeDtypeStruct((_N, _K), jnp.float32),
            jax.ShapeDtypeStruct((_N, _K), jnp.int32),
        ],
        compiler_params=pltpu.CompilerParams(
            dimension_semantics=("arbitrary",),
        ),
    )(norm_bf16, norm_bf16)
    return vals, idxs


# speedup vs baseline: 2.2744x; 1.3666x over previous
import jax
import jax.numpy as jnp
from jax.experimental import pallas as pl
from jax.experimental.pallas import tpu as pltpu

_N = 10000
_D = 128
_K = 20
_RB = 200
_NBLK = _N // _RB
_NEG = -3.0


def _prep_body(x_ref, o_ref):
    x = x_ref[...]
    s = jnp.sum(x * x, axis=1, keepdims=True)
    y = x / jnp.sqrt(jnp.maximum(s, 1e-12))
    o_ref[...] = y.astype(jnp.bfloat16)


def _topk_body(rows_ref, all_ref, vals_ref, idxs_ref):
    i = pl.program_id(0)
    a = rows_ref[...]
    b = all_ref[...]
    sim = jax.lax.dot_general(
        a, b, (((1,), (1,)), ((), ())), preferred_element_type=jnp.float32
    )
    col = jax.lax.broadcasted_iota(jnp.int32, (_RB, _N), 1)
    row = jax.lax.broadcasted_iota(jnp.int32, (_RB, _N), 0) + i * _RB
    s = jnp.where(col == row, -2.0, sim)
    vals = []
    idxs = []
    for _ in range(_K):
        m = jnp.max(s, axis=1, keepdims=True)
        idx = jnp.min(jnp.where(s == m, col, _N), axis=1, keepdims=True)
        vals.append(m)
        idxs.append(idx)
        s = jnp.where(col == idx, _NEG, s)
    vals_ref[...] = jnp.concatenate(vals, axis=1)
    idxs_ref[...] = jnp.concatenate(idxs, axis=1)


def kernel(embeddings):
    norm_bf16 = pl.pallas_call(
        _prep_body,
        out_shape=jax.ShapeDtypeStruct((_N, _D), jnp.bfloat16),
    )(embeddings)

    vals, idxs = pl.pallas_call(
        _topk_body,
        grid=(_NBLK,),
        in_specs=[
            pl.BlockSpec((_RB, _D), lambda i: (i, 0)),
            pl.BlockSpec((_N, _D), lambda i: (0, 0)),
        ],
        out_specs=[
            pl.BlockSpec((_RB, _K), lambda i: (i, 0)),
            pl.BlockSpec((_RB, _K), lambda i: (i, 0)),
        ],
        out_shape=[
            jax.ShapeDtypeStruct((_N, _K), jnp.float32),
            jax.ShapeDtypeStruct((_N, _K), jnp.int32),
        ],
        compiler_params=pltpu.CompilerParams(
            dimension_semantics=("arbitrary",),
        ),
    )(norm_bf16, norm_bf16)
    return vals, idxs


# speedup vs baseline: 2.5955x; 1.1412x over previous
import jax
import jax.numpy as jnp
from jax.experimental import pallas as pl
from jax.experimental.pallas import tpu as pltpu

_N = 10000
_D = 128
_K = 20
_RB = 400
_NBLK = _N // _RB
_NEG = -3.0


def _prep_body(x_ref, o_ref):
    x = x_ref[...]
    s = jnp.sum(x * x, axis=1, keepdims=True)
    y = x / jnp.sqrt(jnp.maximum(s, 1e-12))
    o_ref[...] = y.astype(jnp.bfloat16)


def _topk_body(rows_ref, all_ref, vals_ref, idxs_ref):
    i = pl.program_id(0)
    a = rows_ref[...]
    b = all_ref[...]
    sim = jax.lax.dot_general(
        a, b, (((1,), (1,)), ((), ())), preferred_element_type=jnp.float32
    )
    col = jax.lax.broadcasted_iota(jnp.int32, (_RB, _N), 1)
    row = jax.lax.broadcasted_iota(jnp.int32, (_RB, _N), 0) + i * _RB
    s = jnp.where(col == row, -2.0, sim)
    vals = []
    idxs = []
    for _ in range(_K):
        m = jnp.max(s, axis=1, keepdims=True)
        idx = jnp.min(jnp.where(s == m, col, _N), axis=1, keepdims=True)
        vals.append(m)
        idxs.append(idx)
        s = jnp.where(col == idx, _NEG, s)
    vals_ref[...] = jnp.concatenate(vals, axis=1)
    idxs_ref[...] = jnp.concatenate(idxs, axis=1)


def kernel(embeddings):
    norm_bf16 = pl.pallas_call(
        _prep_body,
        out_shape=jax.ShapeDtypeStruct((_N, _D), jnp.bfloat16),
    )(embeddings)

    vals, idxs = pl.pallas_call(
        _topk_body,
        grid=(_NBLK,),
        in_specs=[
            pl.BlockSpec((_RB, _D), lambda i: (i, 0)),
            pl.BlockSpec((_N, _D), lambda i: (0, 0)),
        ],
        out_specs=[
            pl.BlockSpec((_RB, _K), lambda i: (i, 0)),
            pl.BlockSpec((_RB, _K), lambda i: (i, 0)),
        ],
        out_shape=[
            jax.ShapeDtypeStruct((_N, _K), jnp.float32),
            jax.ShapeDtypeStruct((_N, _K), jnp.int32),
        ],
        compiler_params=pltpu.CompilerParams(
            dimension_semantics=("arbitrary",),
        ),
    )(norm_bf16, norm_bf16)
    return vals, idxs
